# Initial kernel scaffold; baseline (speedup 1.0000x reference)
import jax
import jax.numpy as jnp
from jax.experimental import pallas as pl


def kernel(features, support, mask, W, b):
    raise NotImplementedError("write your pallas kernel here")


# speedup vs baseline: 1.5567x; 1.5567x over previous
import jax
import jax.numpy as jnp
from jax.experimental import pallas as pl

_BLK = 256


def _x_kernel(f_ref, w_ref, b_ref, x_ref, sqrow_ref):
    x = jnp.dot(f_ref[0], w_ref[...], preferred_element_type=jnp.float32)
    x = jnp.maximum(x + b_ref[...], 0.0)
    x_ref[0] = x
    sqrow_ref[0] = jnp.sum(x * x, axis=-1)[None, :]


def _g2_kernel(s_ref, xfull_ref, xrow_ref, sqrow_ref, mrow_ref, mcol_ref,
               out_ref):
    s = s_ref[0]
    mj = mrow_ref[0]
    adj = jnp.where(s > 0.0, 1.0, 0.0) * mj
    deg0 = jnp.sum(adj, axis=1, keepdims=True)
    t2 = jnp.sum(adj * sqrow_ref[0], axis=1, keepdims=True)
    agg = jnp.dot(adj, xfull_ref[0], preferred_element_type=jnp.float32)
    xr = xrow_ref[0]
    sqr = jnp.sum(xr * xr, axis=1, keepdims=True)
    t3 = jnp.sum(xr * agg, axis=1, keepdims=True)
    mi = mcol_ref[0]
    deg = mi * deg0
    diff = mi * (sqr * deg0 + t2 - 2.0 * t3)
    out_ref[0] = jnp.tanh(diff / jnp.maximum(deg, 1.0))


def kernel(features, support, mask, W, b):
    B, N, D = features.shape

    X, sqrow = pl.pallas_call(
        _x_kernel,
        grid=(B,),
        in_specs=[
            pl.BlockSpec((1, N, D), lambda bb: (bb, 0, 0)),
            pl.BlockSpec((D, D), lambda bb: (0, 0)),
            pl.BlockSpec((1, D), lambda bb: (0, 0)),
        ],
        out_specs=[
            pl.BlockSpec((1, N, D), lambda bb: (bb, 0, 0)),
            pl.BlockSpec((1, 1, N), lambda bb: (bb, 0, 0)),
        ],
        out_shape=[
            jax.ShapeDtypeStruct((B, N, D), jnp.float32),
            jax.ShapeDtypeStruct((B, 1, N), jnp.float32),
        ],
    )(features, W, b.reshape(1, D))

    mrow = mask.reshape(B, 1, N)
    tau = pl.pallas_call(
        _g2_kernel,
        grid=(B, N // _BLK),
        in_specs=[
            pl.BlockSpec((1, _BLK, N), lambda bb, i: (bb, i, 0)),
            pl.BlockSpec((1, N, D), lambda bb, i: (bb, 0, 0)),
            pl.BlockSpec((1, _BLK, D), lambda bb, i: (bb, i, 0)),
            pl.BlockSpec((1, 1, N), lambda bb, i: (bb, 0, 0)),
            pl.BlockSpec((1, 1, N), lambda bb, i: (bb, 0, 0)),
            pl.BlockSpec((1, _BLK, 1), lambda bb, i: (bb, i, 0)),
        ],
        out_specs=pl.BlockSpec((1, _BLK, 1), lambda bb, i: (bb, i, 0)),
        out_shape=jax.ShapeDtypeStruct((B, N, 1), jnp.float32),
    )(support, X, X, sqrow, mrow, mask)
    return tau
